# initial kernel scaffold (unmeasured)
import jax
import jax.numpy as jnp
from jax import lax
from jax.experimental import pallas as pl
from jax.experimental.pallas import tpu as pltpu


def kernel(
    x,
):
    def body(*refs):
        pass

    out_shape = jax.ShapeDtypeStruct(..., jnp.float32)
    return pl.pallas_call(body, out_shape=out_shape)(...)


# baseline (device time: 97398 ns/iter reference)
import jax
import jax.numpy as jnp
from jax import lax
from jax.experimental import pallas as pl
from jax.experimental.pallas import tpu as pltpu

N_DEV = 32
R_HOPS = N_DEV // 2
L_HOPS = N_DEV // 2 - 1


def kernel(x):
    m, n = x.shape

    def body(x_ref, out_ref, send_r, recv_r, send_l, recv_l):
        me = lax.axis_index("i")
        right = lax.rem(me + 1, N_DEV)
        left = lax.rem(me - 1 + N_DEV, N_DEV)

        barrier = pltpu.get_barrier_semaphore()
        for nbr in (left, right):
            pl.semaphore_signal(
                barrier, inc=1,
                device_id=(nbr,), device_id_type=pl.DeviceIdType.MESH,
            )
        pl.semaphore_wait(barrier, 2)

        out_ref[pl.ds(me * m, m), :] = x_ref[:, :].astype(out_ref.dtype)

        for h in range(R_HOPS):
            o_r = lax.rem(me - h + N_DEV, N_DEV)
            rd_r = pltpu.make_async_remote_copy(
                src_ref=out_ref.at[pl.ds(o_r * m, m)],
                dst_ref=out_ref.at[pl.ds(o_r * m, m)],
                send_sem=send_r.at[h],
                recv_sem=recv_r.at[h],
                device_id=(right,),
                device_id_type=pl.DeviceIdType.MESH,
            )
            rd_r.start()
            if h < L_HOPS:
                o_l = lax.rem(me + h, N_DEV)
                rd_l = pltpu.make_async_remote_copy(
                    src_ref=out_ref.at[pl.ds(o_l * m, m)],
                    dst_ref=out_ref.at[pl.ds(o_l * m, m)],
                    send_sem=send_l.at[h],
                    recv_sem=recv_l.at[h],
                    device_id=(left,),
                    device_id_type=pl.DeviceIdType.MESH,
                )
                rd_l.start()
            rd_r.wait()
            if h < L_HOPS:
                rd_l.wait()

    return pl.pallas_call(
        body,
        out_shape=jax.ShapeDtypeStruct((N_DEV * m, n), jnp.bfloat16),
        in_specs=[pl.BlockSpec(memory_space=pltpu.VMEM)],
        out_specs=pl.BlockSpec(memory_space=pltpu.VMEM),
        scratch_shapes=[
            pltpu.SemaphoreType.DMA((R_HOPS,)),
            pltpu.SemaphoreType.DMA((R_HOPS,)),
            pltpu.SemaphoreType.DMA((L_HOPS,)),
            pltpu.SemaphoreType.DMA((L_HOPS,)),
        ],
        compiler_params=pltpu.CompilerParams(collective_id=0),
    )(x)


# device time: 65614 ns/iter; 1.4844x vs baseline; 1.4844x over previous
import os as _os

import jax
import jax.numpy as jnp
from jax import lax
from jax.experimental import pallas as pl
from jax.experimental.pallas import tpu as pltpu

if _os.environ.get("KERNEL_PROBE"):
    import distributed_mesh_v7x as _dm
    _mesh = _dm.get_mesh("i", world_size=32)
    for _i, _d in enumerate(_mesh.devices.flat):
        print("LOGICAL", _i, tuple(_d.coords), _d.core_on_chip)

N_DEV = 32
R_HOPS = N_DEV // 2
L_HOPS = N_DEV // 2 - 1

_PLANE = [(0, 0), (1, 0), (1, 1), (0, 1), (0, 2), (1, 2), (1, 3), (0, 3)]
_LOGICAL = {}
for _z in range(4):
    for _j, (_x, _y) in enumerate(_PLANE):
        _LOGICAL[(_x, _y, _z)] = 8 * _z + _j

_P = [(0, 0), (1, 0), (2, 0), (3, 0), (3, 1), (2, 1), (1, 1), (0, 1),
      (0, 2), (1, 2), (2, 2), (3, 2), (3, 3), (2, 3), (1, 3), (0, 3)]
_RING = [(0, y, z) for (y, z) in _P] + [(1, y, z) for (y, z) in reversed(_P)]
PERM = [_LOGICAL[c] for c in _RING]
INV = [0] * N_DEV
for _k, _l in enumerate(PERM):
    INV[_l] = _k


def kernel(x):
    m, n = x.shape

    perm = jnp.array(PERM, dtype=jnp.int32)
    inv = jnp.array(INV, dtype=jnp.int32)
    me = lax.axis_index("i")
    k = inv[me]
    right = perm[(k + 1) % N_DEV]
    left = perm[(k - 1) % N_DEV]
    o_r = perm[(k - jnp.arange(R_HOPS)) % N_DEV]
    o_l = perm[(k + jnp.arange(L_HOPS)) % N_DEV]
    params = jnp.concatenate([jnp.stack([right, left]), o_r, o_l]).astype(
        jnp.int32
    )

    def body(x_ref, params_ref, out_ref, send_r, recv_r, send_l, recv_l):
        my_id = lax.axis_index("i")
        right_id = params_ref[0]
        left_id = params_ref[1]

        barrier = pltpu.get_barrier_semaphore()
        for nbr in (left_id, right_id):
            pl.semaphore_signal(
                barrier, inc=1,
                device_id=(nbr,), device_id_type=pl.DeviceIdType.MESH,
            )
        pl.semaphore_wait(barrier, 2)

        out_ref[pl.ds(my_id * m, m), :] = x_ref[:, :].astype(out_ref.dtype)

        pending = []
        for h in range(R_HOPS):
            o_r_h = params_ref[2 + h]
            rd_r = pltpu.make_async_remote_copy(
                src_ref=out_ref.at[pl.ds(o_r_h * m, m)],
                dst_ref=out_ref.at[pl.ds(o_r_h * m, m)],
                send_sem=send_r.at[h],
                recv_sem=recv_r.at[h],
                device_id=(right_id,),
                device_id_type=pl.DeviceIdType.MESH,
            )
            rd_r.start()
            pending.append(rd_r)
            if h < L_HOPS:
                o_l_h = params_ref[2 + R_HOPS + h]
                rd_l = pltpu.make_async_remote_copy(
                    src_ref=out_ref.at[pl.ds(o_l_h * m, m)],
                    dst_ref=out_ref.at[pl.ds(o_l_h * m, m)],
                    send_sem=send_l.at[h],
                    recv_sem=recv_l.at[h],
                    device_id=(left_id,),
                    device_id_type=pl.DeviceIdType.MESH,
                )
                rd_l.start()
                pending.append(rd_l)
                rd_l.wait_recv()
            rd_r.wait_recv()
        for rd in pending:
            rd.wait_send()

    return pl.pallas_call(
        body,
        out_shape=jax.ShapeDtypeStruct((N_DEV * m, n), jnp.bfloat16),
        in_specs=[
            pl.BlockSpec(memory_space=pltpu.VMEM),
            pl.BlockSpec(memory_space=pltpu.SMEM),
        ],
        out_specs=pl.BlockSpec(memory_space=pltpu.VMEM),
        scratch_shapes=[
            pltpu.SemaphoreType.DMA((R_HOPS,)),
            pltpu.SemaphoreType.DMA((R_HOPS,)),
            pltpu.SemaphoreType.DMA((L_HOPS,)),
            pltpu.SemaphoreType.DMA((L_HOPS,)),
        ],
        compiler_params=pltpu.CompilerParams(collective_id=0),
    )(x, params)


# device time: 50886 ns/iter; 1.9140x vs baseline; 1.2894x over previous
import os as _os

import jax
import jax.numpy as jnp
from jax import lax
from jax.experimental import pallas as pl
from jax.experimental.pallas import tpu as pltpu

if _os.environ.get("KERNEL_PROBE"):
    import distributed_mesh_v7x as _dm
    _mesh = _dm.get_mesh("i", world_size=32)
    for _i, _d in enumerate(_mesh.devices.flat):
        print("LOGICAL", _i, tuple(_d.coords), _d.core_on_chip)

N_DEV = 32
R_HOPS = N_DEV // 2
L_HOPS = N_DEV // 2 - 1

_PLANE = [(0, 0), (1, 0), (1, 1), (0, 1), (0, 2), (1, 2), (1, 3), (0, 3)]
_LOGICAL = {}
for _z in range(4):
    for _j, (_x, _y) in enumerate(_PLANE):
        _LOGICAL[(_x, _y, _z)] = 8 * _z + _j

_P = [(0, 0), (1, 0), (2, 0), (3, 0), (3, 1), (2, 1), (1, 1), (0, 1),
      (0, 2), (1, 2), (2, 2), (3, 2), (3, 3), (2, 3), (1, 3), (0, 3)]
_RING = [(0, y, z) for (y, z) in _P] + [(1, y, z) for (y, z) in reversed(_P)]
PERM = [_LOGICAL[c] for c in _RING]
INV = [0] * N_DEV
for _k, _l in enumerate(PERM):
    INV[_l] = _k


def kernel(x):
    m, n = x.shape

    perm = jnp.array(PERM, dtype=jnp.int32)
    inv = jnp.array(INV, dtype=jnp.int32)
    me = lax.axis_index("i")
    k = inv[me]
    right = perm[(k + 1) % N_DEV]
    left = perm[(k - 1) % N_DEV]
    o_r = perm[(k - jnp.arange(R_HOPS)) % N_DEV]
    o_l = perm[(k + jnp.arange(L_HOPS)) % N_DEV]
    params = jnp.concatenate([jnp.stack([right, left]), o_r, o_l]).astype(
        jnp.int32
    )

    S = int(_os.environ.get("KERNEL_SUBCHUNKS", "4"))
    ms = m // S

    def body(x_ref, params_ref, out_ref, send_r, recv_r, send_l, recv_l):
        my_id = lax.axis_index("i")
        right_id = params_ref[0]
        left_id = params_ref[1]

        barrier = pltpu.get_barrier_semaphore()
        for nbr in (left_id, right_id):
            pl.semaphore_signal(
                barrier, inc=1,
                device_id=(nbr,), device_id_type=pl.DeviceIdType.MESH,
            )
        pl.semaphore_wait(barrier, 2)

        out_ref[pl.ds(my_id * m, m), :] = x_ref[:, :].astype(out_ref.dtype)

        def make(origin, s, send_sems, recv_sems, h, dev):
            row = origin * m + s * ms
            return pltpu.make_async_remote_copy(
                src_ref=out_ref.at[pl.ds(row, ms)],
                dst_ref=out_ref.at[pl.ds(row, ms)],
                send_sem=send_sems.at[h * S + s],
                recv_sem=recv_sems.at[h * S + s],
                device_id=(dev,),
                device_id_type=pl.DeviceIdType.MESH,
            )

        rds_r = [[None] * S for _ in range(R_HOPS)]
        rds_l = [[None] * S for _ in range(L_HOPS)]
        for h in range(R_HOPS):
            o_r_h = params_ref[2 + h]
            o_l_h = params_ref[2 + R_HOPS + h] if h < L_HOPS else None
            for s in range(S):
                if h > 0:
                    rds_r[h - 1][s].wait_recv()
                rd = make(o_r_h, s, send_r, recv_r, h, right_id)
                rd.start()
                rds_r[h][s] = rd
                if h > 0:
                    rds_r[h - 1][s].wait_send()
            if h < L_HOPS:
                for s in range(S):
                    if h > 0:
                        rds_l[h - 1][s].wait_recv()
                    rd = make(o_l_h, s, send_l, recv_l, h, left_id)
                    rd.start()
                    rds_l[h][s] = rd
                    if h > 0:
                        rds_l[h - 1][s].wait_send()
        for s in range(S):
            rds_r[R_HOPS - 1][s].wait_recv()
            rds_r[R_HOPS - 1][s].wait_send()
            rds_l[L_HOPS - 1][s].wait_recv()
            rds_l[L_HOPS - 1][s].wait_send()

    return pl.pallas_call(
        body,
        out_shape=jax.ShapeDtypeStruct((N_DEV * m, n), jnp.bfloat16),
        in_specs=[
            pl.BlockSpec(memory_space=pltpu.VMEM),
            pl.BlockSpec(memory_space=pltpu.SMEM),
        ],
        out_specs=pl.BlockSpec(memory_space=pltpu.VMEM),
        scratch_shapes=[
            pltpu.SemaphoreType.DMA((R_HOPS * S,)),
            pltpu.SemaphoreType.DMA((R_HOPS * S,)),
            pltpu.SemaphoreType.DMA((L_HOPS * S,)),
            pltpu.SemaphoreType.DMA((L_HOPS * S,)),
        ],
        compiler_params=pltpu.CompilerParams(collective_id=0),
    )(x, params)


# device time: 43869 ns/iter; 2.2202x vs baseline; 1.1600x over previous
import os as _os

import jax
import jax.numpy as jnp
from jax import lax
from jax.experimental import pallas as pl
from jax.experimental.pallas import tpu as pltpu

N_DEV = 32
R_HOPS = N_DEV // 2
L_HOPS = N_DEV // 2 - 1


def _perm(k):
    in_x0 = k < 16
    j = jnp.where(in_x0, k, 31 - k)
    x = jnp.where(in_x0, 0, 1)
    z = j // 4
    r = j % 4
    y = jnp.where(z % 2 == 0, r, 3 - r)
    return 8 * z + 2 * y + jnp.where(y % 2 == 0, x, 1 - x)


def _inv(l):
    z = l // 8
    q = l % 8
    y = q // 2
    x = jnp.where(y % 2 == 0, q % 2, 1 - (q % 2))
    r = jnp.where(z % 2 == 0, y, 3 - y)
    j = 4 * z + r
    return jnp.where(x == 0, j, 31 - j)


def kernel(x):
    m, n = x.shape

    S = int(_os.environ.get("KERNEL_SUBCHUNKS", "4"))
    ms = m // S

    def body(x_ref, out_ref, send_r, recv_r, send_l, recv_l):
        my_id = lax.axis_index("i")
        k = _inv(my_id)
        right_id = _perm((k + 1) % N_DEV)
        left_id = _perm((k + N_DEV - 1) % N_DEV)

        barrier = pltpu.get_barrier_semaphore()
        for nbr in (left_id, right_id):
            pl.semaphore_signal(
                barrier, inc=1,
                device_id=(nbr,), device_id_type=pl.DeviceIdType.MESH,
            )
        pl.semaphore_wait(barrier, 2)

        out_ref[pl.ds(my_id * m, m), :] = x_ref[:, :].astype(out_ref.dtype)

        def make(origin, s, send_sems, recv_sems, h, dev):
            row = origin * m + s * ms
            return pltpu.make_async_remote_copy(
                src_ref=out_ref.at[pl.ds(row, ms)],
                dst_ref=out_ref.at[pl.ds(row, ms)],
                send_sem=send_sems.at[h * S + s],
                recv_sem=recv_sems.at[h * S + s],
                device_id=(dev,),
                device_id_type=pl.DeviceIdType.MESH,
            )

        rds_r = [[None] * S for _ in range(R_HOPS)]
        rds_l = [[None] * S for _ in range(L_HOPS)]
        for h in range(R_HOPS):
            o_r_h = _perm((k + N_DEV - h) % N_DEV)
            o_l_h = _perm((k + h) % N_DEV) if h < L_HOPS else None
            for s in range(S):
                if h > 0:
                    rds_r[h - 1][s].wait_recv()
                rd = make(o_r_h, s, send_r, recv_r, h, right_id)
                rd.start()
                rds_r[h][s] = rd
                if h > 0:
                    rds_r[h - 1][s].wait_send()
            if h < L_HOPS:
                for s in range(S):
                    if h > 0:
                        rds_l[h - 1][s].wait_recv()
                    rd = make(o_l_h, s, send_l, recv_l, h, left_id)
                    rd.start()
                    rds_l[h][s] = rd
                    if h > 0:
                        rds_l[h - 1][s].wait_send()
        for s in range(S):
            rds_r[R_HOPS - 1][s].wait_recv()
            rds_r[R_HOPS - 1][s].wait_send()
            rds_l[L_HOPS - 1][s].wait_recv()
            rds_l[L_HOPS - 1][s].wait_send()

    return pl.pallas_call(
        body,
        out_shape=jax.ShapeDtypeStruct((N_DEV * m, n), jnp.bfloat16),
        in_specs=[pl.BlockSpec(memory_space=pltpu.VMEM)],
        out_specs=pl.BlockSpec(memory_space=pltpu.VMEM),
        scratch_shapes=[
            pltpu.SemaphoreType.DMA((R_HOPS * S,)),
            pltpu.SemaphoreType.DMA((R_HOPS * S,)),
            pltpu.SemaphoreType.DMA((L_HOPS * S,)),
            pltpu.SemaphoreType.DMA((L_HOPS * S,)),
        ],
        compiler_params=pltpu.CompilerParams(collective_id=0),
    )(x)


# device time: 33625 ns/iter; 2.8966x vs baseline; 1.3047x over previous
import os as _os

import jax
import jax.numpy as jnp
from jax import lax
from jax.experimental import pallas as pl
from jax.experimental.pallas import tpu as pltpu

N_DEV = 32
N_PAIR = 16
R_HOPS = 8
L_HOPS = 7

_CY = [(0, 0), (1, 0), (2, 0), (3, 0), (3, 1), (3, 2), (3, 3), (2, 3),
       (2, 2), (2, 1), (1, 1), (1, 2), (1, 3), (0, 3), (0, 2), (0, 1)]
Y16 = [y for y, _ in _CY]
Z16 = [z for _, z in _CY]
P16 = [4 * z + y for y, z in _CY]
INV16 = [0] * N_PAIR
for _t, _p in enumerate(P16):
    INV16[_p] = _t


def _tab(idx, table):
    v = jnp.int32(table[0])
    for i in range(1, len(table)):
        v = jnp.where(idx == i, jnp.int32(table[i]), v)
    return v


def kernel(x):
    m, n = x.shape

    S = int(_os.environ.get("KERNEL_SUBCHUNKS", "2"))
    ms = m // S

    def body(x_ref, out_ref, send_x, recv_x, send_r, recv_r, send_l, recv_l):
        me = lax.axis_index("i")
        z = me // 8
        q = me % 8
        y = q // 2
        xs = jnp.where(y % 2 == 0, q % 2, 1 - (q % 2))
        p_me = 4 * z + y
        t = _tab(p_me, INV16)
        my_par = me % 2
        partner = jnp.bitwise_xor(me, 1)

        def ring_dev(tt):
            yy = _tab(tt, Y16)
            zz = _tab(tt, Z16)
            return 8 * zz + 2 * yy + jnp.where(yy % 2 == 0, xs, 1 - xs)

        right_id = ring_dev((t + 1) % N_PAIR)
        left_id = ring_dev((t + N_PAIR - 1) % N_PAIR)

        barrier = pltpu.get_barrier_semaphore()
        for nbr in (partner, left_id, right_id):
            pl.semaphore_signal(
                barrier, inc=1,
                device_id=(nbr,), device_id_type=pl.DeviceIdType.MESH,
            )
        pl.semaphore_wait(barrier, 3)

        out_ref[pl.ds(me * m, m), :] = x_ref[:, :].astype(out_ref.dtype)

        def desc(row, send_sems, recv_sems, idx, dev):
            return pltpu.make_async_remote_copy(
                src_ref=out_ref.at[pl.ds(row, ms)],
                dst_ref=out_ref.at[pl.ds(row, ms)],
                send_sem=send_sems.at[idx],
                recv_sem=recv_sems.at[idx],
                device_id=(dev,),
                device_id_type=pl.DeviceIdType.MESH,
            )

        def idx(h, l, s):
            return h * 2 * S + l * S + s

        po_r = [_tab((t + N_PAIR - h) % N_PAIR, P16) for h in range(R_HOPS)]
        po_l = [_tab((t + h) % N_PAIR, P16) for h in range(L_HOPS)]
        po_in_r0 = _tab((t + N_PAIR - 1) % N_PAIR, P16)
        po_in_l0 = _tab((t + 1) % N_PAIR, P16)

        rd_x = []
        for s in range(S):
            rd = desc(me * m + s * ms, send_x, recv_x, s, partner)
            rd.start()
            rd_x.append(rd)
            own_idx = idx(0, my_par, s)
            desc(me * m + s * ms, send_r, recv_r, own_idx, right_id).start()
            desc(me * m + s * ms, send_l, recv_l, own_idx, left_id).start()
        for s in range(S):
            rd_x[s].wait_recv()
            p_idx = idx(0, 1 - my_par, s)
            row = partner * m + s * ms
            desc(row, send_r, recv_r, p_idx, right_id).start()
            desc(row, send_l, recv_l, p_idx, left_id).start()

        rds_r = [[[None] * S for _ in range(2)] for _ in range(R_HOPS)]
        rds_l = [[[None] * S for _ in range(2)] for _ in range(L_HOPS)]
        for h in range(1, R_HOPS):
            for l in (0, 1):
                for s in range(S):
                    prev = rds_r[h - 1][l][s]
                    if prev is None:
                        prev = desc((2 * po_in_r0 + l) * m + s * ms,
                                    send_r, recv_r, idx(0, l, s), right_id)
                    prev.wait_recv()
                    rd = desc((2 * po_r[h] + l) * m + s * ms,
                              send_r, recv_r, idx(h, l, s), right_id)
                    rd.start()
                    rds_r[h][l][s] = rd
                    prev.wait_send()
                    if h < L_HOPS:
                        prev = rds_l[h - 1][l][s]
                        if prev is None:
                            prev = desc((2 * po_in_l0 + l) * m + s * ms,
                                        send_l, recv_l, idx(0, l, s), left_id)
                        prev.wait_recv()
                        rd = desc((2 * po_l[h] + l) * m + s * ms,
                                  send_l, recv_l, idx(h, l, s), left_id)
                        rd.start()
                        rds_l[h][l][s] = rd
                        prev.wait_send()

        for l in (0, 1):
            for s in range(S):
                rds_r[R_HOPS - 1][l][s].wait_recv()
                rds_r[R_HOPS - 1][l][s].wait_send()
                rds_l[L_HOPS - 1][l][s].wait_recv()
                rds_l[L_HOPS - 1][l][s].wait_send()
        for s in range(S):
            rd_x[s].wait_send()

    return pl.pallas_call(
        body,
        out_shape=jax.ShapeDtypeStruct((N_DEV * m, n), jnp.bfloat16),
        in_specs=[pl.BlockSpec(memory_space=pltpu.VMEM)],
        out_specs=pl.BlockSpec(memory_space=pltpu.VMEM),
        scratch_shapes=[
            pltpu.SemaphoreType.DMA((S,)),
            pltpu.SemaphoreType.DMA((S,)),
            pltpu.SemaphoreType.DMA((R_HOPS * 2 * S,)),
            pltpu.SemaphoreType.DMA((R_HOPS * 2 * S,)),
            pltpu.SemaphoreType.DMA((L_HOPS * 2 * S,)),
            pltpu.SemaphoreType.DMA((L_HOPS * 2 * S,)),
        ],
        compiler_params=pltpu.CompilerParams(collective_id=0),
    )(x)
